# baseline (speedup 1.0000x reference)
import functools

import jax
import jax.numpy as jnp
from jax import lax
from jax.experimental import pallas as pl
from jax.experimental.pallas import tpu as pltpu
from jax.experimental.pallas import tpu_sc as plsc

VOCAB = 1000000
D = 32
H = 50
B = 16384
NW = 32
CB = 1024
GW = 128
N_UNITS = H * (B // CB)
UNITS_PER_W = N_UNITS // NW


def _emb_lookup(table, idx_hm):
    mesh = plsc.VectorSubcoreMesh(core_axis_name="c", subcore_axis_name="s")

    @functools.partial(
        pl.kernel,
        out_type=jax.ShapeDtypeStruct((H, D, B), jnp.float32),
        mesh=mesh,
        scratch_types=[
            pltpu.VMEM((CB,), jnp.int32),
            pltpu.VMEM((CB, D), jnp.float32),
            pltpu.VMEM((D * CB,), jnp.float32),
            pltpu.SemaphoreType.DMA,
        ],
        compiler_params=pltpu.CompilerParams(
            use_tc_tiling_on_sc=False, needs_layout_passes=False
        ),
    )
    def k(table_hbm, idx_hbm, out_hbm, idx_v, rows_v, obuf, sem):
        wid = lax.axis_index("s") * 2 + lax.axis_index("c")
        iota = lax.iota(jnp.int32, 16)

        def unit_body(u, carry):
            unit = u * NW + wid
            h = unit // (B // CB)
            bt8 = unit % (B // CB)
            off = h * B + bt8 * CB
            pltpu.sync_copy(idx_hbm.at[pl.ds(off, CB)], idx_v)
            handles = []
            for j in range(CB // GW):
                handles.append(
                    pltpu.async_copy(
                        table_hbm.at[idx_v.at[pl.ds(j * GW, GW)]],
                        rows_v.at[pl.ds(j * GW, GW)],
                        sem,
                    )
                )
            for hd in handles:
                hd.wait()

            def tr_body(i0, carry2):
                row_idx = i0 + iota
                for c in range(D):
                    col_idx = jnp.full((16,), c, jnp.int32)
                    v = plsc.load_gather(rows_v, [row_idx, col_idx])
                    obuf[pl.ds(c * CB + i0, 16)] = v
                return carry2

            lax.fori_loop(0, CB // 16, lambda i, cc: tr_body(i * 16, cc), 0)

            for c in range(D):
                pltpu.sync_copy(
                    obuf.at[pl.ds(c * CB, CB)],
                    out_hbm.at[h, c, pl.ds(bt8 * CB, CB)],
                )
            return carry

        lax.fori_loop(0, UNITS_PER_W, unit_body, 0)

    return k(table, idx_hm)


def kernel(text, table):
    idx_hm = text.T.reshape(-1).astype(jnp.int32)
    out_hcb = _emb_lookup(table, idx_hm)
    return out_hcb.transpose(2, 0, 1)


# speedup vs baseline: 1.2030x; 1.2030x over previous
import functools

import jax
import jax.numpy as jnp
from jax import lax
from jax.experimental import pallas as pl
from jax.experimental.pallas import tpu as pltpu
from jax.experimental.pallas import tpu_sc as plsc

VOCAB = 1000000
D = 32
H = 50
B = 16384
NW = 32
CB = 1024
GW = 128
N_UNITS = H * (B // CB)
UNITS_PER_W = N_UNITS // NW


def _emb_lookup(table, textT):
    mesh = plsc.VectorSubcoreMesh(core_axis_name="c", subcore_axis_name="s")

    @functools.partial(
        pl.kernel,
        out_type=jax.ShapeDtypeStruct((H, D, B), jnp.float32),
        mesh=mesh,
        scratch_types=[
            pltpu.VMEM((CB,), jnp.int32),
            pltpu.VMEM((CB,), jnp.int32),
            pltpu.VMEM((CB, D), jnp.float32),
            pltpu.VMEM((CB, D), jnp.float32),
            pltpu.VMEM((D, CB), jnp.float32),
            pltpu.SemaphoreType.DMA,
            pltpu.SemaphoreType.DMA,
            pltpu.SemaphoreType.DMA,
        ],
        compiler_params=pltpu.CompilerParams(
            use_tc_tiling_on_sc=False, needs_layout_passes=False
        ),
    )
    def k(table_hbm, textT_hbm, out_hbm, idx0, idx1, rows0, rows1, obuf,
          gsem0, gsem1, wsem):
        wid = lax.axis_index("s") * 2 + lax.axis_index("c")
        iota = lax.iota(jnp.int32, 16)
        cvec0 = iota
        cvec1 = iota + 16

        idx_bufs = (idx0, idx1)
        row_bufs = (rows0, rows1)
        gsems = (gsem0, gsem1)

        def unit_hb(u):
            unit = u * NW + wid
            return unit // (B // CB), unit % (B // CB)

        def fire(u):
            h, bt8 = unit_hb(u)
            s = u % 2
            pltpu.sync_copy(
                textT_hbm.at[h, pl.ds(bt8 * CB, CB)], idx_bufs[s]
            )
            return [
                pltpu.async_copy(
                    table_hbm.at[idx_bufs[s].at[pl.ds(j * GW, GW)]],
                    row_bufs[s].at[pl.ds(j * GW, GW)],
                    gsems[s],
                )
                for j in range(CB // GW)
            ]

        def transpose(rows):
            def body(i0, carry):
                for ii in range(8):
                    i = i0 * 8 + ii
                    iv = jnp.full((16,), 0, jnp.int32) + i
                    v0 = rows[i, pl.ds(0, 16)]
                    v1 = rows[i, pl.ds(16, 16)]
                    plsc.store_scatter(obuf, [cvec0, iv], v0)
                    plsc.store_scatter(obuf, [cvec1, iv], v1)
                return carry

            lax.fori_loop(0, CB // 8, body, 0)

        gh = fire(0)
        wh = []
        for u in range(UNITS_PER_W):
            if u + 1 < UNITS_PER_W:
                gh_next = fire(u + 1)
            else:
                gh_next = []
            for hd in gh:
                hd.wait()
            for hd in wh:
                hd.wait()
            transpose(row_bufs[u % 2])
            h, bt8 = unit_hb(u)
            wh = [
                pltpu.async_copy(
                    obuf,
                    out_hbm.at[h, :, pl.ds(bt8 * CB, CB)],
                    wsem,
                )
            ]
            gh = gh_next
        for hd in wh:
            hd.wait()

    return k(table, textT)


def kernel(text, table):
    textT = text.T.astype(jnp.int32)
    out_hcb = _emb_lookup(table, textT)
    return out_hcb.transpose(2, 0, 1)


# speedup vs baseline: 1.2878x; 1.0705x over previous
import functools

import jax
import jax.numpy as jnp
from jax import lax
from jax.experimental import pallas as pl
from jax.experimental.pallas import tpu as pltpu
from jax.experimental.pallas import tpu_sc as plsc

VOCAB = 1000000
D = 32
H = 50
B = 16384
NW = 32
CB = 1024
GW = 128
N_UNITS = H * (B // CB)
UNITS_PER_W = N_UNITS // NW
BW = B // NW

_MESH = plsc.VectorSubcoreMesh(core_axis_name="c", subcore_axis_name="s")
_SC_PARAMS = pltpu.CompilerParams(
    use_tc_tiling_on_sc=False, needs_layout_passes=False
)


def _flatten_idx(textT):
    @functools.partial(
        pl.kernel,
        out_type=jax.ShapeDtypeStruct((H * B,), jnp.int32),
        mesh=_MESH,
        scratch_types=[pltpu.VMEM((8, BW), jnp.int32)],
        compiler_params=pltpu.CompilerParams(
            use_tc_tiling_on_sc=True, needs_layout_passes=False
        ),
    )
    def k0(textT_hbm, out_hbm, buf):
        wid = lax.axis_index("s") * 2 + lax.axis_index("c")
        b0 = wid * BW
        for band in range(H // 8 + 1):
            nh = min(8, H - band * 8)
            pltpu.sync_copy(
                textT_hbm.at[pl.ds(band * 8, nh), pl.ds(b0, BW)],
                buf.at[pl.ds(0, nh)],
            )
            for hl in range(nh):
                h = band * 8 + hl
                pltpu.sync_copy(
                    buf.at[hl], out_hbm.at[pl.ds(h * B + b0, BW)]
                )

    return k0(textT)


def _emb_lookup(table, idx_hm):
    @functools.partial(
        pl.kernel,
        out_type=jax.ShapeDtypeStruct((H, D, B), jnp.float32),
        mesh=_MESH,
        scratch_types=[
            pltpu.VMEM((CB,), jnp.int32),
            pltpu.VMEM((CB,), jnp.int32),
            pltpu.VMEM((CB, D), jnp.float32),
            pltpu.VMEM((CB, D), jnp.float32),
            pltpu.VMEM((D, CB), jnp.float32),
            pltpu.SemaphoreType.DMA,
            pltpu.SemaphoreType.DMA,
            pltpu.SemaphoreType.DMA,
        ],
        compiler_params=_SC_PARAMS,
    )
    def k(table_hbm, idx_hbm, out_hbm, idx0, idx1, rows0, rows1, obuf,
          gsem0, gsem1, wsem):
        wid = lax.axis_index("s") * 2 + lax.axis_index("c")
        iota = lax.iota(jnp.int32, 16)
        cvec0 = iota
        cvec1 = iota + 16

        idx_bufs = (idx0, idx1)
        row_bufs = (rows0, rows1)
        gsems = (gsem0, gsem1)

        def unit_hb(u):
            unit = u * NW + wid
            return unit // (B // CB), unit % (B // CB)

        def fire(u):
            h, bt8 = unit_hb(u)
            s = u % 2
            pltpu.sync_copy(
                idx_hbm.at[pl.ds(h * B + bt8 * CB, CB)], idx_bufs[s]
            )
            return [
                pltpu.async_copy(
                    table_hbm.at[idx_bufs[s].at[pl.ds(j * GW, GW)]],
                    row_bufs[s].at[pl.ds(j * GW, GW)],
                    gsems[s],
                )
                for j in range(CB // GW)
            ]

        def transpose(rows):
            @plsc.parallel_loop(0, CB, 1, unroll=8)
            def body(i):
                iv = jnp.full((16,), 0, jnp.int32) + i
                v0 = rows[i, pl.ds(0, 16)]
                v1 = rows[i, pl.ds(16, 16)]
                plsc.store_scatter(obuf, [cvec0, iv], v0)
                plsc.store_scatter(obuf, [cvec1, iv], v1)

        gh = fire(0)
        wh = []
        for u in range(UNITS_PER_W):
            if u + 1 < UNITS_PER_W:
                gh_next = fire(u + 1)
            else:
                gh_next = []
            for hd in gh:
                hd.wait()
            for hd in wh:
                hd.wait()
            transpose(row_bufs[u % 2])
            h, bt8 = unit_hb(u)
            wh = [
                pltpu.async_copy(
                    obuf,
                    out_hbm.at[h, :, pl.ds(bt8 * CB, CB)],
                    wsem,
                )
            ]
            gh = gh_next
        for hd in wh:
            hd.wait()

    return k(table, idx_hm)


def kernel(text, table):
    textT = text.T.astype(jnp.int32)
    idx_hm = _flatten_idx(textT)
    out_hcb = _emb_lookup(table, idx_hm)
    return out_hcb.transpose(2, 0, 1)


# speedup vs baseline: 1.9652x; 1.5260x over previous
import functools

import jax
import jax.numpy as jnp
from jax import lax
from jax.experimental import pallas as pl
from jax.experimental.pallas import tpu as pltpu
from jax.experimental.pallas import tpu_sc as plsc

VOCAB = 1000000
D = 32
H = 50
B = 16384
NW = 32
CB = 1024
GW = 128
N_UNITS = H * (B // CB)
UNITS_PER_W = N_UNITS // NW
BW = B // NW

_MESH = plsc.VectorSubcoreMesh(core_axis_name="c", subcore_axis_name="s")
_SC_PARAMS = pltpu.CompilerParams(
    use_tc_tiling_on_sc=False, needs_layout_passes=False
)


def _flatten_idx(textT):
    @functools.partial(
        pl.kernel,
        out_type=jax.ShapeDtypeStruct((H * B,), jnp.int32),
        mesh=_MESH,
        scratch_types=[pltpu.VMEM((8, BW), jnp.int32)],
        compiler_params=pltpu.CompilerParams(
            use_tc_tiling_on_sc=True, needs_layout_passes=False
        ),
    )
    def k0(textT_hbm, out_hbm, buf):
        wid = lax.axis_index("s") * 2 + lax.axis_index("c")
        b0 = wid * BW
        for band in range(H // 8 + 1):
            nh = min(8, H - band * 8)
            pltpu.sync_copy(
                textT_hbm.at[pl.ds(band * 8, nh), pl.ds(b0, BW)],
                buf.at[pl.ds(0, nh)],
            )
            for hl in range(nh):
                h = band * 8 + hl
                pltpu.sync_copy(
                    buf.at[hl], out_hbm.at[pl.ds(h * B + b0, BW)]
                )

    return k0(textT)


def _emb_lookup(table, idx_hm):
    @functools.partial(
        pl.kernel,
        out_type=jax.ShapeDtypeStruct((H, D, B), jnp.float32),
        mesh=_MESH,
        scratch_types=[
            pltpu.VMEM((CB,), jnp.int32),
            pltpu.VMEM((CB,), jnp.int32),
            pltpu.VMEM((CB, D), jnp.float32),
            pltpu.VMEM((CB, D), jnp.float32),
            pltpu.VMEM((D, CB + 17), jnp.float32),
            pltpu.SemaphoreType.DMA,
            pltpu.SemaphoreType.DMA,
            pltpu.SemaphoreType.DMA,
        ],
        compiler_params=_SC_PARAMS,
    )
    def k(table_hbm, idx_hbm, out_hbm, idx0, idx1, rows0, rows1, obuf,
          gsem0, gsem1, wsem):
        wid = lax.axis_index("s") * 2 + lax.axis_index("c")
        iota = lax.iota(jnp.int32, 16)
        cvec0 = iota
        cvec1 = iota + 16

        idx_bufs = (idx0, idx1)
        row_bufs = (rows0, rows1)
        gsems = (gsem0, gsem1)

        def unit_hb(u):
            unit = u * NW + wid
            return unit // (B // CB), unit % (B // CB)

        def fire(u):
            h, bt8 = unit_hb(u)
            s = u % 2
            pltpu.sync_copy(
                idx_hbm.at[pl.ds(h * B + bt8 * CB, CB)], idx_bufs[s]
            )
            return [
                pltpu.async_copy(
                    table_hbm.at[idx_bufs[s].at[pl.ds(j * GW, GW)]],
                    row_bufs[s].at[pl.ds(j * GW, GW)],
                    gsems[s],
                )
                for j in range(CB // GW)
            ]

        def transpose(rows):
            @plsc.parallel_loop(0, CB, 1, unroll=8)
            def body(i):
                iv = jnp.full((16,), 0, jnp.int32) + i
                v0 = rows[i, pl.ds(0, 16)]
                v1 = rows[i, pl.ds(16, 16)]
                plsc.store_scatter(obuf, [cvec0, iv], v0)
                plsc.store_scatter(obuf, [cvec1, iv], v1)

        gh = fire(0)
        wh = []
        for u in range(UNITS_PER_W):
            if u + 1 < UNITS_PER_W:
                gh_next = fire(u + 1)
            else:
                gh_next = []
            for hd in gh:
                hd.wait()
            for hd in wh:
                hd.wait()
            transpose(row_bufs[u % 2])
            h, bt8 = unit_hb(u)
            wh = [
                pltpu.async_copy(
                    obuf.at[:, pl.ds(0, CB)],
                    out_hbm.at[h, :, pl.ds(bt8 * CB, CB)],
                    wsem,
                )
            ]
            gh = gh_next
        for hd in wh:
            hd.wait()

    return k(table, idx_hm)


def kernel(text, table):
    textT = text.T.astype(jnp.int32)
    idx_hm = _flatten_idx(textT)
    out_hcb = _emb_lookup(table, idx_hm)
    return out_hcb.transpose(2, 0, 1)
